# Initial kernel scaffold; baseline (speedup 1.0000x reference)
import jax
import jax.numpy as jnp
from jax.experimental import pallas as pl


def kernel(x, w1, b1, g2, w2, b2, g3, w3, b3, wl1, bl1, wl2, bl2, wha, bha):
    raise NotImplementedError("write your pallas kernel here")


# speedup vs baseline: 9.0110x; 9.0110x over previous
import functools

import jax
import jax.numpy as jnp
from jax import lax
from jax.experimental import pallas as pl
from jax.experimental.pallas import tpu as pltpu

_K1, _S1 = 8, 4
_K2, _S2 = 4, 2
_K3, _S3 = 3, 1


def _conv_out(sz, k, s):
    return (sz - k) // s + 1


def _im2col_conv1(x, dtype):
    B, C, H, W = x.shape
    k, s = _K1, _S1
    oh, ow = _conv_out(H, k, s), _conv_out(W, k, s)
    xb = x.astype(dtype)
    cols = []
    for i in range(k):
        for j in range(k):
            cols.append(lax.slice(xb, (0, 0, i, j),
                                  (B, C, i + s * (oh - 1) + 1, j + s * (ow - 1) + 1),
                                  (1, 1, s, s)))
    pt = jnp.stack(cols, axis=0).reshape(k, k, B, C, oh, ow)
    pt = pt.transpose(2, 4, 5, 3, 0, 1)
    pt = pt.reshape(B, oh * ow, C * k * k)
    pt = pt.reshape(B, oh // 2, 2, ow // 2, 2, C * k * k)
    pt = pt.transpose(0, 2, 4, 1, 3, 5)
    return pt.reshape(B, oh * ow, C * k * k)


def _fwd_kernel(patches_ref, w1_ref, b1_ref, w2_ref, b2_ref, w3_ref, b3_ref,
                wl1_ref, bl1_ref, wl2_ref, bl2_ref, wha_ref, bha_ref, out_ref,
                *, TB, O1H, O1W, O2H, O2W, O3H, O3W, nA):
    f32 = jnp.float32
    w1 = w1_ref[...]
    mmdt = w1.dtype

    x = patches_ref[...]
    h1 = jnp.dot(x, w1, preferred_element_type=f32) + b1_ref[...]
    h1 = jnp.maximum(h1, 0.0).astype(mmdt)
    C1 = h1.shape[1]
    h1 = h1.reshape(TB, 4, O1H // 2, O1W // 2, C1)

    taps2 = [h1[:, (i % 2) * 2 + (j % 2),
                i // 2:i // 2 + O2H, j // 2:j // 2 + O2W, :]
             for i in range(_K2) for j in range(_K2)]
    g2 = jnp.concatenate(taps2, axis=3).reshape(TB * O2H * O2W, _K2 * _K2 * C1)
    h2 = jnp.dot(g2, w2_ref[...], preferred_element_type=f32) + b2_ref[...]
    h2 = jnp.maximum(h2, 0.0).astype(mmdt)
    C2 = h2.shape[1]
    h2 = h2.reshape(TB, O2H, O2W, C2)

    taps3 = [h2[:, i:i + O3H, j:j + O3W, :]
             for i in range(_K3) for j in range(_K3)]
    g3 = jnp.concatenate(taps3, axis=3).reshape(TB * O3H * O3W, _K3 * _K3 * C2)
    h3 = jnp.dot(g3, w3_ref[...], preferred_element_type=f32) + b3_ref[...]
    h3 = jnp.maximum(h3, 0.0)
    C3 = h3.shape[1]

    h3 = h3.reshape(TB, O3H * O3W, C3).astype(mmdt)
    flat = jnp.concatenate([h3[:, sp, :] for sp in range(O3H * O3W)],
                           axis=1)
    f1 = jnp.dot(flat, wl1_ref[...], preferred_element_type=f32) + bl1_ref[...]
    f1 = jnp.maximum(f1, 0.0).astype(mmdt)
    f2 = jnp.dot(f1, wl2_ref[...], preferred_element_type=f32) + bl2_ref[...]

    qv = jnp.dot(f2.astype(mmdt), wha_ref[...], preferred_element_type=f32) + bha_ref[...]
    a = qv[:, :nA]
    v = qv[:, nA:nA + 1]
    q = a + (v - jnp.sum(a, axis=1, keepdims=True) * (1.0 / nA))
    out_ref[...] = q.astype(out_ref.dtype)


def kernel(x, w1, b1, g2, w2, b2, g3, w3, b3, wl1, bl1, wl2, bl2, wha, bha):
    del g2, g3
    B = x.shape[0]
    H, W = x.shape[2], x.shape[3]
    o1h, o1w = _conv_out(H, _K1, _S1), _conv_out(W, _K1, _S1)
    o2h, o2w = _conv_out(o1h, _K2, _S2), _conv_out(o1w, _K2, _S2)
    o3h, o3w = _conv_out(o2h, _K3, _S3), _conv_out(o2w, _K3, _S3)
    nA = wha.shape[1] - 1
    K1dim = w1.shape[0]
    P1 = o1h * o1w

    TB = 16
    while B % TB:
        TB //= 2
    nblk = B // TB

    patches = _im2col_conv1(x, w1.dtype).reshape(B * P1, K1dim)

    weights = (w1, b1, w2, b2, w3, b3, wl1, bl1, wl2, bl2, wha, bha)
    in_specs = [pl.BlockSpec((TB * P1, K1dim), lambda n: (n, 0))]
    in_specs += [pl.BlockSpec(a.shape, lambda n, _nd=a.ndim: (0,) * _nd)
                 for a in weights]
    out_specs = pl.BlockSpec((TB, nA), lambda n: (n, 0))

    C1, C2, C3 = w1.shape[1], w2.shape[1], w3.shape[1]
    N1, N2 = wl1.shape[1], wl2.shape[1]
    p2, p3 = o2h * o2w, o3h * o3w
    flops = 2 * B * (P1 * K1dim * C1 + p2 * _K2 * _K2 * C1 * C2
                     + p3 * _K3 * _K3 * C2 * C3 + p3 * C3 * N1
                     + N1 * N2 + N2 * (nA + 1))
    bytes_accessed = (patches.size * patches.dtype.itemsize
                      + sum(a.size * a.dtype.itemsize for a in weights)
                      + B * nA * 4)

    kern = functools.partial(_fwd_kernel, TB=TB, O1H=o1h, O1W=o1w,
                             O2H=o2h, O2W=o2w, O3H=o3h, O3W=o3w, nA=nA)
    out = pl.pallas_call(
        kern,
        out_shape=jax.ShapeDtypeStruct((B, nA), jnp.float32),
        grid=(nblk,),
        in_specs=in_specs,
        out_specs=out_specs,
        compiler_params=pltpu.CompilerParams(
            dimension_semantics=("parallel",)),
        cost_estimate=pl.CostEstimate(flops=int(flops), transcendentals=0,
                                      bytes_accessed=int(bytes_accessed)),
    )(patches, *weights)
    return out
